# baseline (speedup 1.0000x reference)
import functools

import jax
import jax.numpy as jnp
from jax import lax
from jax.experimental import pallas as pl
from jax.experimental.pallas import tpu as pltpu
from jax.experimental.pallas import tpu_sc as plsc

N_G, N_E, N_U = 8192, 2048, 512
DEG = 8
F = 42
D = 128
BLOCKS = 4
NC, NS = 2, 16
NW = NC * NS


@functools.lru_cache(maxsize=None)
def _chunk(per_w):
    return 128 if per_w >= 512 else max(16, per_w // 4)


def _make_sc_edge(n, E):
    per_w = E // NW
    K = _chunk(per_w)
    steps = per_w // K
    NSLOT = min(3, steps)
    rows_per_tile = n // NS
    resident = 3 * n * D * 4 <= 2 * 2**20
    mesh = plsc.VectorSubcoreMesh(core_axis_name="c", subcore_axis_name="s")

    table_types = (
        [pltpu.VMEM_SHARED((n, D), jnp.float32)] * 2 if resident else [])

    @functools.partial(
        pl.kernel,
        out_type=jax.ShapeDtypeStruct((NC, n, D), jnp.float32),
        mesh=mesh,
        scratch_types=[
            pltpu.VMEM((steps, K), jnp.int32),
            pltpu.VMEM((steps, K), jnp.int32),
            pltpu.VMEM((NSLOT, K, D), jnp.float32),
            pltpu.VMEM_SHARED((n, D), jnp.float32),
        ] + table_types + [
            pltpu.SemaphoreType.DMA((NSLOT,)),
            pltpu.SemaphoreType.DMA((NSLOT,)),
            pltpu.SemaphoreType.DMA((NSLOT,)),
            pltpu.SemaphoreType.DMA((8,)),
        ],
    )
    def sc_edge(a_hbm, b_hbm, src_hbm, dst_hbm, out_hbm,
                src_v, dst_v, m_v, agg_sh, *rest):
        if resident:
            a_sh, b_sh, sem_a, sem_b, sem_s, sem_i = rest
        else:
            sem_a, sem_b, sem_s, sem_i = rest
        cid = lax.axis_index("c")
        sid = lax.axis_index("s")
        wid = cid * NS + sid
        r0 = sid * rows_per_tile
        init = []

        row0 = wid * steps
        init.append(pltpu.async_copy(src_hbm.at[pl.ds(row0, steps)], src_v,
                                     sem_i.at[0]))
        init.append(pltpu.async_copy(dst_hbm.at[pl.ds(row0, steps)], dst_v,
                                     sem_i.at[1]))
        if resident:
            init.append(pltpu.async_copy(a_hbm.at[pl.ds(r0, rows_per_tile)],
                                         a_sh.at[pl.ds(r0, rows_per_tile)],
                                         sem_i.at[2]))
            init.append(pltpu.async_copy(b_hbm.at[pl.ds(r0, rows_per_tile)],
                                         b_sh.at[pl.ds(r0, rows_per_tile)],
                                         sem_i.at[3]))

        def zbody(e4, c):
            for u in range(4):
                for j in range(D // 16):
                    m_v[0, e4 * 4 + u, pl.ds(j * 16, 16)] = jnp.zeros(
                        (16,), jnp.float32)
            return c

        lax.fori_loop(0, K // 4, zbody, 0)
        for c in range((rows_per_tile + K - 1) // K):
            rows = min(K, rows_per_tile - c * K)
            init.append(pltpu.async_copy(m_v.at[0, pl.ds(0, rows)],
                                         agg_sh.at[pl.ds(r0 + c * K, rows)],
                                         sem_i.at[4 + c % 4]))
        for cp in init:
            cp.wait()
        plsc.subcore_barrier()
        a_src = a_sh if resident else a_hbm
        b_src = b_sh if resident else b_hbm

        ga = [None] * NSLOT
        gb = [None] * NSLOT
        sc = [None] * NSLOT

        def wait_(lst, s):
            if lst[s] is not None:
                lst[s].wait()
                lst[s] = None

        started = set()

        def ensure_a(i):
            if i not in started:
                slot = i % NSLOT
                wait_(sc, slot)
                ga[slot] = pltpu.async_copy(a_src.at[src_v.at[i]],
                                            m_v.at[slot], sem_a.at[slot])
                started.add(i)

        for j in range(min(NSLOT - 1, steps)):
            ensure_a(j)

        for i in range(steps):
            slot = i % NSLOT
            ensure_a(i)
            wait_(ga, slot)
            gb[slot] = pltpu.async_copy(b_src.at[dst_v.at[i]], m_v.at[slot],
                                        sem_b.at[slot], add=True)
            if i + NSLOT - 1 < steps:
                ensure_a(i + NSLOT - 1)
            wait_(gb, slot)

            def body(e4, c, _slot=slot):
                for u in range(4):
                    for jj in range(D // 16):
                        s = pl.ds(jj * 16, 16)
                        m_v[_slot, e4 * 4 + u, s] = jnp.maximum(
                            m_v[_slot, e4 * 4 + u, s], 0.0)
                return c

            lax.fori_loop(0, K // 4, body, 0)
            wait_(sc, slot)
            sc[slot] = pltpu.async_copy(m_v.at[slot], agg_sh.at[dst_v.at[i]],
                                        sem_s.at[slot], add=True)
        for s in range(NSLOT):
            wait_(sc, s)
        plsc.subcore_barrier()
        pltpu.sync_copy(agg_sh.at[pl.ds(r0, rows_per_tile)],
                        out_hbm.at[cid, pl.ds(r0, rows_per_tile)])

    return sc_edge


def _enc_pre_body(x_ref, w_ref, wab_ref, h_ref, a_ref, b_ref):
    h = jnp.maximum(
        jnp.dot(x_ref[...], w_ref[...], preferred_element_type=jnp.float32), 0.0)
    h_ref[...] = h
    ab = jnp.dot(h, wab_ref[...], preferred_element_type=jnp.float32)
    a_ref[...] = ab[:, :D]
    b_ref[...] = ab[:, D:]


def _enc_pool_pre_body(x_ref, w_ref, wp_ref, hp_ref, wab_ref,
                       h_ref, a_ref, b_ref):
    p = jnp.dot(wp_ref[...], hp_ref[...], preferred_element_type=jnp.float32)
    h = jnp.maximum(
        jnp.dot(x_ref[...], w_ref[...], preferred_element_type=jnp.float32),
        0.0) + p
    h_ref[...] = h
    ab = jnp.dot(h, wab_ref[...], preferred_element_type=jnp.float32)
    a_ref[...] = ab[:, :D]
    b_ref[...] = ab[:, D:]


def _ln(hn):
    mu = jnp.mean(hn, axis=-1, keepdims=True)
    var = jnp.mean((hn - mu) ** 2, axis=-1, keepdims=True)
    return (hn - mu) * lax.rsqrt(var + 1e-5)


def _new_h(h_ref, aggp_ref, wn_ref):
    h = h_ref[...]
    hagg = jnp.concatenate([h, aggp_ref[0] + aggp_ref[1]], axis=1)
    upd = jnp.maximum(
        jnp.dot(hagg, wn_ref[...], preferred_element_type=jnp.float32), 0.0)
    return _ln(h + upd)


def _upd_pre_body(h_ref, aggp_ref, wn_ref, nwab_ref, h_out, a_ref, b_ref):
    hn = _new_h(h_ref, aggp_ref, wn_ref)
    h_out[...] = hn
    ab = jnp.dot(hn, nwab_ref[...], preferred_element_type=jnp.float32)
    a_ref[...] = ab[:, :D]
    b_ref[...] = ab[:, D:]


def _node_upd_body(h_ref, aggp_ref, wn_ref, o_ref):
    o_ref[...] = _new_h(h_ref, aggp_ref, wn_ref)


def _upd_dec_body(h_ref, aggp_ref, wn_ref, wd_ref, o_ref):
    o_ref[...] = jnp.dot(_new_h(h_ref, aggp_ref, wn_ref), wd_ref[...],
                         preferred_element_type=jnp.float32)


def _hab(n):
    return [jax.ShapeDtypeStruct((n, D), jnp.float32)] * 3


def _enc_pre(x, w, wab):
    n = x.shape[0]
    return pl.pallas_call(_enc_pre_body, out_shape=_hab(n))(x, w, wab)


def _enc_pool_pre(x, w, wp, hp, wab):
    n = x.shape[0]
    m_src = wp.shape[1]
    bm = 256
    return pl.pallas_call(
        _enc_pool_pre_body,
        grid=(n // bm,),
        in_specs=[pl.BlockSpec((bm, F), lambda i: (i, 0)),
                  pl.BlockSpec((F, D), lambda i: (0, 0)),
                  pl.BlockSpec((bm, m_src), lambda i: (i, 0)),
                  pl.BlockSpec((m_src, D), lambda i: (0, 0)),
                  pl.BlockSpec((D, 2 * D), lambda i: (0, 0))],
        out_specs=[pl.BlockSpec((bm, D), lambda i: (i, 0))] * 3,
        out_shape=_hab(n))(x, w, wp, hp, wab)


def _upd_pre(h, aggp, wn, nwab):
    n = h.shape[0]
    return pl.pallas_call(
        _upd_pre_body, out_shape=_hab(n))(h, aggp, wn, nwab)


def _node_upd(h, aggp, wn):
    n = h.shape[0]
    return pl.pallas_call(
        _node_upd_body,
        out_shape=jax.ShapeDtypeStruct((n, D), jnp.float32))(h, aggp, wn)


def _upd_dec(h, aggp, wn, wd):
    n = h.shape[0]
    return pl.pallas_call(
        _upd_dec_body,
        out_shape=jax.ShapeDtypeStruct((n, F), jnp.float32))(h, aggp, wn, wd)


def _run_level(x, edge, W_enc, W_edge, W_node, n, pool=None, W_dec=None):
    E = edge.shape[1]
    K = _chunk(E // NW)
    src = edge[0].reshape(E // K, K)
    dst = edge[1].reshape(E // K, K)
    sc_edge = _make_sc_edge(n, E)
    Wab = jnp.concatenate([W_edge[:, :D, :], W_edge[:, D:, :]], axis=2)
    if pool is None:
        h, a, bb = _enc_pre(x, W_enc, Wab[0])
    else:
        h, a, bb = _enc_pool_pre(x, W_enc, pool[0], pool[1], Wab[0])
    for b in range(BLOCKS):
        aggp = sc_edge(a, bb, src, dst)
        if b + 1 < BLOCKS:
            h, a, bb = _upd_pre(h, aggp, W_node[b], Wab[b + 1])
        elif W_dec is not None:
            h = _upd_dec(h, aggp, W_node[b], W_dec)
        else:
            h = _node_upd(h, aggp, W_node[b])
    return h


def kernel(x_global, x_europe, x_uk, edge_global, edge_europe, edge_uk,
           W_enc_g, W_edge_g, W_node_g,
           W_enc_e, W_edge_e, W_node_e,
           W_enc_u, W_edge_u, W_node_u,
           W_pool1, W_pool2, W_dec):
    out_g = _run_level(x_global, edge_global, W_enc_g, W_edge_g, W_node_g, N_G)
    out_e = _run_level(x_europe, edge_europe, W_enc_e, W_edge_e, W_node_e,
                       N_E, pool=(W_pool1, out_g))
    return _run_level(x_uk, edge_uk, W_enc_u, W_edge_u, W_node_u,
                      N_U, pool=(W_pool2, out_e), W_dec=W_dec)


# speedup vs baseline: 1.0058x; 1.0058x over previous
import functools

import jax
import jax.numpy as jnp
from jax import lax
from jax.experimental import pallas as pl
from jax.experimental.pallas import tpu as pltpu
from jax.experimental.pallas import tpu_sc as plsc

N_G, N_E, N_U = 8192, 2048, 512
DEG = 8
F = 42
D = 128
BLOCKS = 4
NC, NS = 2, 16
NW = NC * NS


@functools.lru_cache(maxsize=None)
def _chunk(per_w):
    return 128 if per_w >= 512 else max(16, per_w // 4)


def _make_sc_edge(n, E):
    per_w = E // NW
    K = _chunk(per_w)
    steps = per_w // K
    NSLOT = min(3, steps)
    rows_per_tile = n // NS
    resident = 3 * n * D * 4 <= 7 * 2**20
    mesh = plsc.VectorSubcoreMesh(core_axis_name="c", subcore_axis_name="s")

    table_types = (
        [pltpu.VMEM_SHARED((n, D), jnp.float32)] * 2 if resident else [])

    @functools.partial(
        pl.kernel,
        out_type=jax.ShapeDtypeStruct((NC, n, D), jnp.float32),
        mesh=mesh,
        scratch_types=[
            pltpu.VMEM((steps, K), jnp.int32),
            pltpu.VMEM((steps, K), jnp.int32),
            pltpu.VMEM((NSLOT, K, D), jnp.float32),
            pltpu.VMEM_SHARED((n, D), jnp.float32),
        ] + table_types + [
            pltpu.SemaphoreType.DMA((NSLOT,)),
            pltpu.SemaphoreType.DMA((NSLOT,)),
            pltpu.SemaphoreType.DMA((NSLOT,)),
            pltpu.SemaphoreType.DMA((8,)),
        ],
    )
    def sc_edge(a_hbm, b_hbm, src_hbm, dst_hbm, out_hbm,
                src_v, dst_v, m_v, agg_sh, *rest):
        if resident:
            a_sh, b_sh, sem_a, sem_b, sem_s, sem_i = rest
        else:
            sem_a, sem_b, sem_s, sem_i = rest
        cid = lax.axis_index("c")
        sid = lax.axis_index("s")
        wid = cid * NS + sid
        r0 = sid * rows_per_tile
        init = []

        row0 = wid * steps
        init.append(pltpu.async_copy(src_hbm.at[pl.ds(row0, steps)], src_v,
                                     sem_i.at[0]))
        init.append(pltpu.async_copy(dst_hbm.at[pl.ds(row0, steps)], dst_v,
                                     sem_i.at[1]))
        if resident:
            init.append(pltpu.async_copy(a_hbm.at[pl.ds(r0, rows_per_tile)],
                                         a_sh.at[pl.ds(r0, rows_per_tile)],
                                         sem_i.at[2]))
            init.append(pltpu.async_copy(b_hbm.at[pl.ds(r0, rows_per_tile)],
                                         b_sh.at[pl.ds(r0, rows_per_tile)],
                                         sem_i.at[3]))

        def zbody(e4, c):
            for u in range(4):
                for j in range(D // 16):
                    m_v[0, e4 * 4 + u, pl.ds(j * 16, 16)] = jnp.zeros(
                        (16,), jnp.float32)
            return c

        lax.fori_loop(0, K // 4, zbody, 0)
        for c in range((rows_per_tile + K - 1) // K):
            rows = min(K, rows_per_tile - c * K)
            init.append(pltpu.async_copy(m_v.at[0, pl.ds(0, rows)],
                                         agg_sh.at[pl.ds(r0 + c * K, rows)],
                                         sem_i.at[4 + c % 4]))
        for cp in init:
            cp.wait()
        plsc.subcore_barrier()
        a_src = a_sh if resident else a_hbm
        b_src = b_sh if resident else b_hbm

        ga = [None] * NSLOT
        gb = [None] * NSLOT
        sc = [None] * NSLOT

        def wait_(lst, s):
            if lst[s] is not None:
                lst[s].wait()
                lst[s] = None

        started = set()

        def ensure_a(i):
            if i not in started:
                slot = i % NSLOT
                wait_(sc, slot)
                ga[slot] = pltpu.async_copy(a_src.at[src_v.at[i]],
                                            m_v.at[slot], sem_a.at[slot])
                started.add(i)

        for j in range(min(NSLOT - 1, steps)):
            ensure_a(j)

        for i in range(steps):
            slot = i % NSLOT
            ensure_a(i)
            wait_(ga, slot)
            gb[slot] = pltpu.async_copy(b_src.at[dst_v.at[i]], m_v.at[slot],
                                        sem_b.at[slot], add=True)
            if i + NSLOT - 1 < steps:
                ensure_a(i + NSLOT - 1)
            wait_(gb, slot)

            def body(e4, c, _slot=slot):
                for u in range(4):
                    for jj in range(D // 16):
                        s = pl.ds(jj * 16, 16)
                        m_v[_slot, e4 * 4 + u, s] = jnp.maximum(
                            m_v[_slot, e4 * 4 + u, s], 0.0)
                return c

            lax.fori_loop(0, K // 4, body, 0)
            wait_(sc, slot)
            sc[slot] = pltpu.async_copy(m_v.at[slot], agg_sh.at[dst_v.at[i]],
                                        sem_s.at[slot], add=True)
        for s in range(NSLOT):
            wait_(sc, s)
        plsc.subcore_barrier()
        pltpu.sync_copy(agg_sh.at[pl.ds(r0, rows_per_tile)],
                        out_hbm.at[cid, pl.ds(r0, rows_per_tile)])

    return sc_edge


def _enc_pre_body(x_ref, w_ref, wab_ref, h_ref, a_ref, b_ref):
    h = jnp.maximum(
        jnp.dot(x_ref[...], w_ref[...], preferred_element_type=jnp.float32), 0.0)
    h_ref[...] = h
    ab = jnp.dot(h, wab_ref[...], preferred_element_type=jnp.float32)
    a_ref[...] = ab[:, :D]
    b_ref[...] = ab[:, D:]


def _enc_pool_pre_body(x_ref, w_ref, wp_ref, hp_ref, wab_ref,
                       h_ref, a_ref, b_ref):
    p = jnp.dot(wp_ref[...], hp_ref[...], preferred_element_type=jnp.float32)
    h = jnp.maximum(
        jnp.dot(x_ref[...], w_ref[...], preferred_element_type=jnp.float32),
        0.0) + p
    h_ref[...] = h
    ab = jnp.dot(h, wab_ref[...], preferred_element_type=jnp.float32)
    a_ref[...] = ab[:, :D]
    b_ref[...] = ab[:, D:]


def _ln(hn):
    mu = jnp.mean(hn, axis=-1, keepdims=True)
    var = jnp.mean((hn - mu) ** 2, axis=-1, keepdims=True)
    return (hn - mu) * lax.rsqrt(var + 1e-5)


def _new_h(h_ref, aggp_ref, wn_ref):
    h = h_ref[...]
    hagg = jnp.concatenate([h, aggp_ref[0] + aggp_ref[1]], axis=1)
    upd = jnp.maximum(
        jnp.dot(hagg, wn_ref[...], preferred_element_type=jnp.float32), 0.0)
    return _ln(h + upd)


def _upd_pre_body(h_ref, aggp_ref, wn_ref, nwab_ref, h_out, a_ref, b_ref):
    hn = _new_h(h_ref, aggp_ref, wn_ref)
    h_out[...] = hn
    ab = jnp.dot(hn, nwab_ref[...], preferred_element_type=jnp.float32)
    a_ref[...] = ab[:, :D]
    b_ref[...] = ab[:, D:]


def _node_upd_body(h_ref, aggp_ref, wn_ref, o_ref):
    o_ref[...] = _new_h(h_ref, aggp_ref, wn_ref)


def _upd_dec_body(h_ref, aggp_ref, wn_ref, wd_ref, o_ref):
    o_ref[...] = jnp.dot(_new_h(h_ref, aggp_ref, wn_ref), wd_ref[...],
                         preferred_element_type=jnp.float32)


def _hab(n):
    return [jax.ShapeDtypeStruct((n, D), jnp.float32)] * 3


def _enc_pre(x, w, wab):
    n = x.shape[0]
    return pl.pallas_call(_enc_pre_body, out_shape=_hab(n))(x, w, wab)


def _enc_pool_pre(x, w, wp, hp, wab):
    n = x.shape[0]
    m_src = wp.shape[1]
    bm = 256
    return pl.pallas_call(
        _enc_pool_pre_body,
        grid=(n // bm,),
        in_specs=[pl.BlockSpec((bm, F), lambda i: (i, 0)),
                  pl.BlockSpec((F, D), lambda i: (0, 0)),
                  pl.BlockSpec((bm, m_src), lambda i: (i, 0)),
                  pl.BlockSpec((m_src, D), lambda i: (0, 0)),
                  pl.BlockSpec((D, 2 * D), lambda i: (0, 0))],
        out_specs=[pl.BlockSpec((bm, D), lambda i: (i, 0))] * 3,
        out_shape=_hab(n))(x, w, wp, hp, wab)


def _upd_pre(h, aggp, wn, nwab):
    n = h.shape[0]
    return pl.pallas_call(
        _upd_pre_body, out_shape=_hab(n))(h, aggp, wn, nwab)


def _node_upd(h, aggp, wn):
    n = h.shape[0]
    return pl.pallas_call(
        _node_upd_body,
        out_shape=jax.ShapeDtypeStruct((n, D), jnp.float32))(h, aggp, wn)


def _upd_dec(h, aggp, wn, wd):
    n = h.shape[0]
    return pl.pallas_call(
        _upd_dec_body,
        out_shape=jax.ShapeDtypeStruct((n, F), jnp.float32))(h, aggp, wn, wd)


def _run_level(x, edge, W_enc, W_edge, W_node, n, pool=None, W_dec=None):
    E = edge.shape[1]
    K = _chunk(E // NW)
    src = edge[0].reshape(E // K, K)
    dst = edge[1].reshape(E // K, K)
    sc_edge = _make_sc_edge(n, E)
    Wab = jnp.concatenate([W_edge[:, :D, :], W_edge[:, D:, :]], axis=2)
    if pool is None:
        h, a, bb = _enc_pre(x, W_enc, Wab[0])
    else:
        h, a, bb = _enc_pool_pre(x, W_enc, pool[0], pool[1], Wab[0])
    for b in range(BLOCKS):
        aggp = sc_edge(a, bb, src, dst)
        if b + 1 < BLOCKS:
            h, a, bb = _upd_pre(h, aggp, W_node[b], Wab[b + 1])
        elif W_dec is not None:
            h = _upd_dec(h, aggp, W_node[b], W_dec)
        else:
            h = _node_upd(h, aggp, W_node[b])
    return h


def kernel(x_global, x_europe, x_uk, edge_global, edge_europe, edge_uk,
           W_enc_g, W_edge_g, W_node_g,
           W_enc_e, W_edge_e, W_node_e,
           W_enc_u, W_edge_u, W_node_u,
           W_pool1, W_pool2, W_dec):
    out_g = _run_level(x_global, edge_global, W_enc_g, W_edge_g, W_node_g, N_G)
    out_e = _run_level(x_europe, edge_europe, W_enc_e, W_edge_e, W_node_e,
                       N_E, pool=(W_pool1, out_g))
    return _run_level(x_uk, edge_uk, W_enc_u, W_edge_u, W_node_u,
                      N_U, pool=(W_pool2, out_e), W_dec=W_dec)
